# Initial kernel scaffold; baseline (speedup 1.0000x reference)
import jax
import jax.numpy as jnp
from jax.experimental import pallas as pl


def kernel(query, keys, values):
    raise NotImplementedError("write your pallas kernel here")


# speedup vs baseline: 1.6149x; 1.6149x over previous
import functools

import jax
import jax.numpy as jnp
from jax import lax
from jax.experimental import pallas as pl
from jax.experimental.pallas import tpu as pltpu
from jax.experimental.pallas import tpu_sc as plsc

B = 1024
D = 64
N = 100000
K = 8
BLK = 2048
NPAD = 102400
NBLK = NPAD // BLK
NEG = -3.0e38

NW = 32
BPW = (B * K) // NW
QPW = B // NW


def _topk_body(q_ref, k_ref, wexp_ref, idx_ref, bv_ref, bi_ref):
    j = pl.program_id(0)

    @pl.when(j == 0)
    def _():
        bv_ref[...] = jnp.full((B, K), NEG, jnp.float32)
        bi_ref[...] = jnp.zeros((B, K), jnp.int32)

    q = q_ref[...]
    qn = q / jnp.maximum(jnp.sqrt(jnp.sum(q * q, axis=1, keepdims=True)), 1e-12)
    kb = k_ref[...]
    kn = kb / jnp.maximum(jnp.sqrt(jnp.sum(kb * kb, axis=1, keepdims=True)), 1e-12)
    sim = lax.dot_general(qn, kn, (((1,), (1,)), ((), ())),
                          preferred_element_type=jnp.float32)

    col = lax.broadcasted_iota(jnp.int32, (B, BLK), 1)
    base = j * BLK
    sim = jnp.where(col + base < N, sim, NEG)

    bvs, bis = [], []
    for _t in range(K):
        m = jnp.max(sim, axis=1, keepdims=True)
        am = jnp.min(jnp.where(sim == m, col, BLK), axis=1, keepdims=True)
        bvs.append(m)
        bis.append(am + base)
        sim = jnp.where(col == am, NEG, sim)
    blk_v = jnp.concatenate(bvs, axis=1)
    blk_i = jnp.concatenate(bis, axis=1)

    cv = jnp.concatenate([bv_ref[...], blk_v], axis=1)
    ci = jnp.concatenate([bi_ref[...], blk_i], axis=1)
    col16 = lax.broadcasted_iota(jnp.int32, (B, 2 * K), 1)
    nvs, nis = [], []
    for _t in range(K):
        m = jnp.max(cv, axis=1, keepdims=True)
        am = jnp.min(jnp.where(cv == m, col16, 2 * K), axis=1, keepdims=True)
        sel = jnp.sum(jnp.where(col16 == am, ci, 0), axis=1, keepdims=True)
        nvs.append(m)
        nis.append(sel)
        cv = jnp.where(col16 == am, NEG, cv)
    bv_ref[...] = jnp.concatenate(nvs, axis=1)
    bi_ref[...] = jnp.concatenate(nis, axis=1)

    @pl.when(j == NBLK - 1)
    def _():
        bv = bv_ref[...]
        e = jnp.exp(bv - bv[:, 0:1])
        w = e / jnp.sum(e, axis=1, keepdims=True)
        wexp_ref[...] = jnp.concatenate(
            [jnp.broadcast_to(w[:, t:t + 1], (B, D)) for t in range(K)], axis=1)
        idx_ref[...] = bi_ref[...]


def _topk_call(query, keys_pad):
    return pl.pallas_call(
        _topk_body,
        grid=(NBLK,),
        in_specs=[
            pl.BlockSpec((B, D), lambda j: (0, 0)),
            pl.BlockSpec((BLK, D), lambda j: (j, 0)),
        ],
        out_specs=[
            pl.BlockSpec((B, K * D), lambda j: (0, 0)),
            pl.BlockSpec((B, K), lambda j: (0, 0)),
        ],
        out_shape=[
            jax.ShapeDtypeStruct((B, K * D), jnp.float32),
            jax.ShapeDtypeStruct((B, K), jnp.int32),
        ],
        scratch_shapes=[
            pltpu.VMEM((B, K), jnp.float32),
            pltpu.VMEM((B, K), jnp.int32),
        ],
        compiler_params=pltpu.CompilerParams(
            dimension_semantics=("arbitrary",)),
    )(query, keys_pad)


def _sc_gather_body(values_hbm, idx_hbm, wexp_hbm, out_hbm,
                    idx_v, rows_v, w_v, out_v, sem):
    wid = lax.axis_index("s") * 2 + lax.axis_index("c")
    base = wid * BPW
    pltpu.sync_copy(idx_hbm.at[pl.ds(base, BPW)], idx_v)
    cps = [pltpu.async_copy(values_hbm.at[idx_v.at[pl.ds(h * 128, 128)]],
                            rows_v.at[pl.ds(h * 128, 128)], sem)
           for h in range(BPW // 128)]
    pltpu.sync_copy(wexp_hbm.at[pl.ds(base, BPW)], w_v)
    for cp in cps:
        cp.wait()

    @pl.loop(0, QPW)
    def _(q):
        r0 = q * K
        for c in range(D // 16):
            cs = pl.ds(c * 16, 16)
            acc = (rows_v.at[pl.ds(r0, 1), cs][...]
                   * w_v.at[pl.ds(r0, 1), cs][...])
            for k in range(1, K):
                acc = acc + (rows_v.at[pl.ds(r0 + k, 1), cs][...]
                             * w_v.at[pl.ds(r0 + k, 1), cs][...])
            out_v.at[pl.ds(q, 1), cs][...] = acc

    pltpu.sync_copy(out_v, out_hbm.at[pl.ds(wid * QPW, QPW)])


@functools.lru_cache(maxsize=1)
def _sc_gather_kernel():
    return pl.kernel(
        _sc_gather_body,
        mesh=plsc.VectorSubcoreMesh(core_axis_name="c", subcore_axis_name="s"),
        out_type=jax.ShapeDtypeStruct((B, D), jnp.float32),
        scratch_types=[
            pltpu.VMEM((BPW,), jnp.int32),
            pltpu.VMEM((BPW, D), jnp.float32),
            pltpu.VMEM((BPW, D), jnp.float32),
            pltpu.VMEM((QPW, D), jnp.float32),
            pltpu.SemaphoreType.DMA,
        ],
        compiler_params=pltpu.CompilerParams(use_tc_tiling_on_sc=False),
    )


def kernel(query, keys, values):
    keys_pad = jnp.pad(keys, ((0, NPAD - N), (0, 0)))
    wexp, idx = _topk_call(query, keys_pad)
    idx_flat = idx.reshape(B * K)
    wexp_flat = wexp.reshape(B, K, D).reshape(B * K, D)
    return _sc_gather_kernel()(values, idx_flat, wexp_flat)


# speedup vs baseline: 2.0803x; 1.2882x over previous
import functools

import jax
import jax.numpy as jnp
from jax import lax
from jax.experimental import pallas as pl
from jax.experimental.pallas import tpu as pltpu
from jax.experimental.pallas import tpu_sc as plsc

B = 1024
D = 64
N = 100000
K = 8
BLK = 2048
NBLK = -(-N // BLK)
NEG = -3.0e38

NW = 32
BPW = (B * K) // NW
QPW = B // NW


def _topk_body(q_ref, k_ref, wexp_ref, idx_ref, bv_ref, bi_ref):
    j = pl.program_id(0)

    @pl.when(j == 0)
    def _():
        bv_ref[...] = jnp.full((B, K), NEG, jnp.float32)
        bi_ref[...] = jnp.zeros((B, K), jnp.int32)

    q = q_ref[...]
    qn = q / jnp.maximum(jnp.sqrt(jnp.sum(q * q, axis=1, keepdims=True)), 1e-12)
    kb = k_ref[...]
    kn = kb / jnp.maximum(jnp.sqrt(jnp.sum(kb * kb, axis=1, keepdims=True)), 1e-12)
    sim = lax.dot_general(qn, kn, (((1,), (1,)), ((), ())),
                          preferred_element_type=jnp.float32)

    col = lax.broadcasted_iota(jnp.int32, (B, BLK), 1)
    base = j * BLK
    sim = jnp.where(col + base < N, sim, NEG)

    k8 = lax.broadcasted_iota(jnp.int32, (B, K), 1)

    def cond(carry):
        sim, m, bv, bi = carry
        return jnp.any(m >= bv[:, K - 1:K])

    def body(carry):
        sim, m, bv, bi = carry
        iseq = sim == m
        am = jnp.min(jnp.where(iseq, col, BLK), axis=1, keepdims=True)
        gidx = am + base
        pos = jnp.sum((bv >= m).astype(jnp.int32), axis=1, keepdims=True)
        bvm1 = jnp.concatenate([bv[:, :1], bv[:, :K - 1]], axis=1)
        bim1 = jnp.concatenate([bi[:, :1], bi[:, :K - 1]], axis=1)
        bv = jnp.where(k8 < pos, bv, jnp.where(k8 == pos, m, bvm1))
        bi = jnp.where(k8 < pos, bi, jnp.where(k8 == pos, gidx, bim1))
        sim = jnp.where(col == am, NEG, sim)
        m = jnp.max(sim, axis=1, keepdims=True)
        return sim, m, bv, bi

    m0 = jnp.max(sim, axis=1, keepdims=True)
    _, _, bv, bi = lax.while_loop(
        cond, body, (sim, m0, bv_ref[...], bi_ref[...]))
    bv_ref[...] = bv
    bi_ref[...] = bi

    @pl.when(j == NBLK - 1)
    def _():
        bv = bv_ref[...]
        e = jnp.exp(bv - bv[:, 0:1])
        w = e / jnp.sum(e, axis=1, keepdims=True)
        wexp_ref[...] = jnp.concatenate(
            [jnp.broadcast_to(w[:, t:t + 1], (B, D)) for t in range(K)], axis=1)
        idx_ref[...] = bi_ref[...]


def _topk_call(query, keys_pad):
    return pl.pallas_call(
        _topk_body,
        grid=(NBLK,),
        in_specs=[
            pl.BlockSpec((B, D), lambda j: (0, 0)),
            pl.BlockSpec((BLK, D), lambda j: (j, 0)),
        ],
        out_specs=[
            pl.BlockSpec((B, K * D), lambda j: (0, 0)),
            pl.BlockSpec((B, K), lambda j: (0, 0)),
        ],
        out_shape=[
            jax.ShapeDtypeStruct((B, K * D), jnp.float32),
            jax.ShapeDtypeStruct((B, K), jnp.int32),
        ],
        scratch_shapes=[
            pltpu.VMEM((B, K), jnp.float32),
            pltpu.VMEM((B, K), jnp.int32),
        ],
        compiler_params=pltpu.CompilerParams(
            dimension_semantics=("arbitrary",)),
    )(query, keys_pad)


def _sc_gather_body(values_hbm, idx_hbm, wexp_hbm, out_hbm,
                    idx_v, rows_v, w_v, out_v, sem):
    wid = lax.axis_index("s") * 2 + lax.axis_index("c")
    base = wid * BPW
    pltpu.sync_copy(idx_hbm.at[pl.ds(base, BPW)], idx_v)
    cps = [pltpu.async_copy(values_hbm.at[idx_v.at[pl.ds(h * 128, 128)]],
                            rows_v.at[pl.ds(h * 128, 128)], sem)
           for h in range(BPW // 128)]
    pltpu.sync_copy(wexp_hbm.at[pl.ds(base, BPW)], w_v)
    for cp in cps:
        cp.wait()

    @pl.loop(0, QPW)
    def _(q):
        r0 = q * K
        for c in range(D // 16):
            cs = pl.ds(c * 16, 16)
            acc = (rows_v.at[pl.ds(r0, 1), cs][...]
                   * w_v.at[pl.ds(r0, 1), cs][...])
            for k in range(1, K):
                acc = acc + (rows_v.at[pl.ds(r0 + k, 1), cs][...]
                             * w_v.at[pl.ds(r0 + k, 1), cs][...])
            out_v.at[pl.ds(q, 1), cs][...] = acc

    pltpu.sync_copy(out_v, out_hbm.at[pl.ds(wid * QPW, QPW)])


@functools.lru_cache(maxsize=1)
def _sc_gather_kernel():
    return pl.kernel(
        _sc_gather_body,
        mesh=plsc.VectorSubcoreMesh(core_axis_name="c", subcore_axis_name="s"),
        out_type=jax.ShapeDtypeStruct((B, D), jnp.float32),
        scratch_types=[
            pltpu.VMEM((BPW,), jnp.int32),
            pltpu.VMEM((BPW, D), jnp.float32),
            pltpu.VMEM((BPW, D), jnp.float32),
            pltpu.VMEM((QPW, D), jnp.float32),
            pltpu.SemaphoreType.DMA,
        ],
        compiler_params=pltpu.CompilerParams(use_tc_tiling_on_sc=False),
    )


def kernel(query, keys, values):
    wexp, idx = _topk_call(query, keys)
    idx_flat = idx.reshape(B * K)
    wexp_flat = wexp.reshape(B, K, D).reshape(B * K, D)
    return _sc_gather_kernel()(values, idx_flat, wexp_flat)


# speedup vs baseline: 2.0931x; 1.0062x over previous
import functools

import jax
import jax.numpy as jnp
from jax import lax
from jax.experimental import pallas as pl
from jax.experimental.pallas import tpu as pltpu
from jax.experimental.pallas import tpu_sc as plsc

B = 1024
D = 64
N = 100000
K = 8
BLK = 2048
NBLK = -(-N // BLK)
NEG = -3.0e38

NW = 32
BPW = (B * K) // NW
QPW = B // NW


def _topk_body(q_ref, k_ref, wexp_ref, idx_ref, bv_ref, bi_ref):
    j = pl.program_id(0)

    @pl.when(j == 0)
    def _():
        bv_ref[...] = jnp.full((B, K), NEG, jnp.float32)
        bi_ref[...] = jnp.zeros((B, K), jnp.int32)

    base = j * BLK
    q = q_ref[...]
    qn = q / jnp.maximum(jnp.sqrt(jnp.sum(q * q, axis=1, keepdims=True)), 1e-12)
    kb = k_ref[...]
    kn = kb / jnp.maximum(jnp.sqrt(jnp.sum(kb * kb, axis=1, keepdims=True)), 1e-12)
    rowv = (lax.broadcasted_iota(jnp.int32, (BLK, 1), 0) + base) < N
    kn1 = jnp.concatenate(
        [jnp.where(rowv, kn, 0.0), jnp.where(rowv, 0.0, -4.0)], axis=1)
    qn1 = jnp.concatenate([qn, jnp.ones((B, 1), jnp.float32)], axis=1)
    sim = lax.dot_general(qn1, kn1, (((1,), (1,)), ((), ())),
                          preferred_element_type=jnp.float32)

    col = lax.broadcasted_iota(jnp.int32, (B, BLK), 1)

    k8 = lax.broadcasted_iota(jnp.int32, (B, K), 1)

    def cond(carry):
        sim, m, bv, bi = carry
        return jnp.any(m >= bv[:, K - 1:K])

    def body(carry):
        sim, m, bv, bi = carry
        iseq = sim == m
        am = jnp.min(jnp.where(iseq, col, BLK), axis=1, keepdims=True)
        gidx = am + base
        pos = jnp.sum((bv >= m).astype(jnp.int32), axis=1, keepdims=True)
        bvm1 = jnp.concatenate([bv[:, :1], bv[:, :K - 1]], axis=1)
        bim1 = jnp.concatenate([bi[:, :1], bi[:, :K - 1]], axis=1)
        bv = jnp.where(k8 < pos, bv, jnp.where(k8 == pos, m, bvm1))
        bi = jnp.where(k8 < pos, bi, jnp.where(k8 == pos, gidx, bim1))
        sim = jnp.where(col == am, NEG, sim)
        m = jnp.max(sim, axis=1, keepdims=True)
        return sim, m, bv, bi

    m0 = jnp.max(sim, axis=1, keepdims=True)
    _, _, bv, bi = lax.while_loop(
        cond, body, (sim, m0, bv_ref[...], bi_ref[...]))
    bv_ref[...] = bv
    bi_ref[...] = bi

    @pl.when(j == NBLK - 1)
    def _():
        bv = bv_ref[...]
        e = jnp.exp(bv - bv[:, 0:1])
        w = e / jnp.sum(e, axis=1, keepdims=True)
        wexp_ref[...] = jnp.concatenate(
            [jnp.broadcast_to(w[:, t:t + 1], (B, D)) for t in range(K)], axis=1)
        idx_ref[...] = bi_ref[...]


def _topk_call(query, keys_pad):
    return pl.pallas_call(
        _topk_body,
        grid=(NBLK,),
        in_specs=[
            pl.BlockSpec((B, D), lambda j: (0, 0)),
            pl.BlockSpec((BLK, D), lambda j: (j, 0)),
        ],
        out_specs=[
            pl.BlockSpec((B, K * D), lambda j: (0, 0)),
            pl.BlockSpec((B, K), lambda j: (0, 0)),
        ],
        out_shape=[
            jax.ShapeDtypeStruct((B, K * D), jnp.float32),
            jax.ShapeDtypeStruct((B, K), jnp.int32),
        ],
        scratch_shapes=[
            pltpu.VMEM((B, K), jnp.float32),
            pltpu.VMEM((B, K), jnp.int32),
        ],
        compiler_params=pltpu.CompilerParams(
            dimension_semantics=("arbitrary",)),
    )(query, keys_pad)


def _sc_gather_body(values_hbm, idx_hbm, wexp_hbm, out_hbm,
                    idx_v, rows_v, w_v, out_v, sem):
    wid = lax.axis_index("s") * 2 + lax.axis_index("c")
    base = wid * BPW
    pltpu.sync_copy(idx_hbm.at[pl.ds(base, BPW)], idx_v)
    cps = [pltpu.async_copy(values_hbm.at[idx_v.at[pl.ds(h * 128, 128)]],
                            rows_v.at[pl.ds(h * 128, 128)], sem)
           for h in range(BPW // 128)]
    pltpu.sync_copy(wexp_hbm.at[pl.ds(wid * QPW, QPW)], w_v)
    for cp in cps:
        cp.wait()

    @pl.loop(0, QPW)
    def _(q):
        r0 = q * K
        for c in range(D // 16):
            cs = pl.ds(c * 16, 16)
            acc = (rows_v.at[pl.ds(r0, 1), cs][...]
                   * w_v.at[pl.ds(q, 1), pl.ds(c * 16, 16)][...])
            for k in range(1, K):
                acc = acc + (rows_v.at[pl.ds(r0 + k, 1), cs][...]
                             * w_v.at[pl.ds(q, 1), pl.ds(k * D + c * 16, 16)][...])
            out_v.at[pl.ds(q, 1), cs][...] = acc

    pltpu.sync_copy(out_v, out_hbm.at[pl.ds(wid * QPW, QPW)])


@functools.lru_cache(maxsize=1)
def _sc_gather_kernel():
    return pl.kernel(
        _sc_gather_body,
        mesh=plsc.VectorSubcoreMesh(core_axis_name="c", subcore_axis_name="s"),
        out_type=jax.ShapeDtypeStruct((B, D), jnp.float32),
        scratch_types=[
            pltpu.VMEM((BPW,), jnp.int32),
            pltpu.VMEM((BPW, D), jnp.float32),
            pltpu.VMEM((QPW, K * D), jnp.float32),
            pltpu.VMEM((QPW, D), jnp.float32),
            pltpu.SemaphoreType.DMA,
        ],
        compiler_params=pltpu.CompilerParams(use_tc_tiling_on_sc=False),
    )


def kernel(query, keys, values):
    wexp, idx = _topk_call(query, keys)
    return _sc_gather_kernel()(values, idx.reshape(B * K), wexp)
